# baseline (speedup 1.0000x reference)
import jax
import jax.numpy as jnp
from jax.experimental import pallas as pl
from jax.experimental.pallas import tpu as pltpu

EOS = 1e-10
BM = 1024
BN = 1024
NSLOT = 3


def _elu1(a):
    return jnp.where(a > 0, a + 1.0, jnp.exp(a))


def _pair_maps(nb):
    im, jm = [], []
    for i in range(nb):
        for j in range(i, nb):
            im.append(i)
            jm.append(j)
    return jnp.array(im, jnp.int32), jnp.array(jm, jnp.int32)


def _pair_maps_main(nb):
    im, jm, ati, atj = [], [], [], []
    prev = None
    for i in range(nb):
        for (a, b) in [(i, j) for j in range(i + 1, nb)] + [(i, i)]:
            im.append(a)
            jm.append(b)
            cur = (b, a) if a != b else (prev if prev is not None else (a, b))
            ati.append(cur[0])
            atj.append(cur[1])
            prev = cur
    return (jnp.array(im, jnp.int32), jnp.array(jm, jnp.int32),
            jnp.array(ati, jnp.int32), jnp.array(atj, jnp.int32))


def _stats_kernel(a_ref, x_ref, w1_ref, b1_ref, rowsum_ref, colpart_ref, h1_ref):
    j = pl.program_id(1)
    e = _elu1(a_ref[:])
    rs = jnp.sum(e, axis=1, keepdims=True)

    @pl.when(j == 0)
    def _():
        rowsum_ref[:] = rs
        h1_ref[:] = (
            jnp.dot(x_ref[:], w1_ref[:], preferred_element_type=jnp.float32)
            + b1_ref[:]
        )

    @pl.when(j != 0)
    def _():
        rowsum_ref[:] += rs

    colpart_ref[:] = jnp.sum(e, axis=0).reshape(1, 1, -1)


def _wait_pair_copies(im_ref, jm_ref, adjn_ref, ab_ref, tb_ref, sema_ref, semb_ref, kk):
    slot = jax.lax.rem(kk, NSLOT)
    i2 = im_ref[kk]
    j2 = jm_ref[kk]
    pltpu.make_async_copy(
        ab_ref.at[slot],
        adjn_ref.at[pl.ds(i2 * BM, BM), pl.ds(j2 * BN, BN)],
        sema_ref.at[slot],
    ).wait()

    @pl.when(i2 != j2)
    def _():
        pltpu.make_async_copy(
            tb_ref.at[slot],
            adjn_ref.at[pl.ds(j2 * BM, BM), pl.ds(i2 * BN, BN)],
            semb_ref.at[slot],
        ).wait()


def _main_kernel(
    im_ref, jm_ref, ati_ref, atj_ref,
    a_ref, at_ref, h1j_ref, h1i_ref, dc_ref, dr_ref,
    adjn_ref, y1_ref, ab_ref, tb_ref, sema_ref, semb_ref,
):
    k = pl.program_id(0)
    npairs = pl.num_programs(0)
    slot = jax.lax.rem(k, NSLOT)
    i = im_ref[k]
    j = jm_ref[k]

    @pl.when(k >= NSLOT)
    def _():
        _wait_pair_copies(
            im_ref, jm_ref, adjn_ref, ab_ref, tb_ref, sema_ref, semb_ref, k - NSLOT
        )

    @pl.when(k == 0)
    def _():
        y1_ref[:] = jnp.zeros_like(y1_ref)

    t = jnp.where(i == j, a_ref[:], at_ref[:])
    e = 0.5 * (_elu1(a_ref[:]) + _elu1(t).T)
    adjn = dc_ref[:] * e * dr_ref[:]
    ab_ref[slot] = adjn
    pltpu.make_async_copy(
        ab_ref.at[slot],
        adjn_ref.at[pl.ds(i * BM, BM), pl.ds(j * BN, BN)],
        sema_ref.at[slot],
    ).start()
    tb_ref[slot] = adjn.T

    @pl.when(i != j)
    def _():
        pltpu.make_async_copy(
            tb_ref.at[slot],
            adjn_ref.at[pl.ds(j * BM, BM), pl.ds(i * BN, BN)],
            semb_ref.at[slot],
        ).start()

    y1_ref[pl.ds(i * BM, BM), :] += jnp.dot(
        adjn, h1j_ref[:], preferred_element_type=jnp.float32
    )

    @pl.when(i != j)
    def _():
        y1_ref[pl.ds(j * BN, BN), :] += jax.lax.dot_general(
            adjn, h1i_ref[:],
            (((0,), (0,)), ((), ())),
            preferred_element_type=jnp.float32,
        )

    @pl.when(k == npairs - 1)
    def _():
        for d in range(NSLOT - 1, 0, -1):
            @pl.when(k >= d)
            def _(d=d):
                _wait_pair_copies(
                    im_ref, jm_ref, adjn_ref, ab_ref, tb_ref, sema_ref, semb_ref,
                    k - d,
                )

        _wait_pair_copies(
            im_ref, jm_ref, adjn_ref, ab_ref, tb_ref, sema_ref, semb_ref, k
        )


def _out_kernel(
    im_ref, jm_ref, ym_ref, adjn_ref, y1j_ref, w2_ref, b2_ref,
    out_ref, h2_ref,
):
    k = pl.program_id(0)
    i = im_ref[k]
    j = jm_ref[k]

    @pl.when(k == 0)
    def _():
        out_ref[:] = jnp.zeros_like(out_ref)

    @pl.when(i == 0)
    def _():
        h = jnp.maximum(y1j_ref[:], 0.0)
        h2_ref[pl.ds(j * BN, BN), :] = (
            jnp.dot(h, w2_ref[:], preferred_element_type=jnp.float32) + b2_ref[:]
        )

    out_ref[pl.ds(i * BM, BM), :] += jnp.dot(
        adjn_ref[:], h2_ref[pl.ds(j * BN, BN), :], preferred_element_type=jnp.float32
    )

    @pl.when(i != j)
    def _():
        out_ref[pl.ds(j * BN, BN), :] += jax.lax.dot_general(
            adjn_ref[:], h2_ref[pl.ds(i * BM, BM), :],
            (((0,), (0,)), ((), ())),
            preferred_element_type=jnp.float32,
        )


def kernel(features, x, Adj_param, W1, b1, W2, b2):
    N = Adj_param.shape[0]
    in_dim = x.shape[1]
    hid = W1.shape[1]
    ncls = W2.shape[1]
    nb = N // BM
    npairs = nb * (nb + 1) // 2
    im, jm = _pair_maps(nb)

    rowsum, colpart, h1 = pl.pallas_call(
        _stats_kernel,
        grid=(nb, nb),
        in_specs=[
            pl.BlockSpec((BM, BN), lambda i, j: (i, j)),
            pl.BlockSpec((BM, in_dim), lambda i, j: (i, 0)),
            pl.BlockSpec((in_dim, hid), lambda i, j: (0, 0)),
            pl.BlockSpec((1, hid), lambda i, j: (0, 0)),
        ],
        out_specs=[
            pl.BlockSpec((BM, 1), lambda i, j: (i, 0)),
            pl.BlockSpec((1, 1, BN), lambda i, j: (i, 0, j)),
            pl.BlockSpec((BM, hid), lambda i, j: (i, 0)),
        ],
        out_shape=[
            jax.ShapeDtypeStruct((N, 1), jnp.float32),
            jax.ShapeDtypeStruct((nb, 1, N), jnp.float32),
            jax.ShapeDtypeStruct((N, hid), jnp.float32),
        ],
    )(Adj_param, x, W1, b1.reshape(1, hid))

    deg = 0.5 * (rowsum[:, 0] + jnp.sum(colpart, axis=(0, 1)))
    dinv = 1.0 / (jnp.sqrt(deg) + EOS)
    dc = dinv[:, None]
    dr = dinv[None, :]

    imm, jmm, ati, atj = _pair_maps_main(nb)
    adjn, y1 = pl.pallas_call(
        _main_kernel,
        grid_spec=pltpu.PrefetchScalarGridSpec(
            num_scalar_prefetch=4,
            grid=(npairs,),
            in_specs=[
                pl.BlockSpec((BM, BN), lambda k, im, jm, ai, aj: (im[k], jm[k])),
                pl.BlockSpec((BN, BM), lambda k, im, jm, ai, aj: (ai[k], aj[k])),
                pl.BlockSpec((BN, hid), lambda k, im, jm, ai, aj: (jm[k], 0)),
                pl.BlockSpec((BM, hid), lambda k, im, jm, ai, aj: (im[k], 0)),
                pl.BlockSpec((BM, 1), lambda k, im, jm, ai, aj: (im[k], 0)),
                pl.BlockSpec((1, BN), lambda k, im, jm, ai, aj: (0, jm[k])),
            ],
            out_specs=[
                pl.BlockSpec(memory_space=pl.ANY),
                pl.BlockSpec((N, hid), lambda k, im, jm, ai, aj: (0, 0)),
            ],
            scratch_shapes=[
                pltpu.VMEM((NSLOT, BM, BN), jnp.float32),
                pltpu.VMEM((NSLOT, BN, BM), jnp.float32),
                pltpu.SemaphoreType.DMA((NSLOT,)),
                pltpu.SemaphoreType.DMA((NSLOT,)),
            ],
        ),
        out_shape=[
            jax.ShapeDtypeStruct((N, N), jnp.float32),
            jax.ShapeDtypeStruct((N, hid), jnp.float32),
        ],
    )(imm, jmm, ati, atj, Adj_param, Adj_param, h1, h1, dc, dr)

    ym = jnp.where(im == 0, jm, nb - 1)
    out = pl.pallas_call(
        _out_kernel,
        grid_spec=pltpu.PrefetchScalarGridSpec(
            num_scalar_prefetch=3,
            grid=(npairs,),
            in_specs=[
                pl.BlockSpec((BM, BN), lambda k, im, jm, ym: (im[k], jm[k])),
                pl.BlockSpec((BN, hid), lambda k, im, jm, ym: (ym[k], 0)),
                pl.BlockSpec((hid, ncls), lambda k, im, jm, ym: (0, 0)),
                pl.BlockSpec((1, ncls), lambda k, im, jm, ym: (0, 0)),
            ],
            out_specs=pl.BlockSpec((N, ncls), lambda k, im, jm, ym: (0, 0)),
            scratch_shapes=[pltpu.VMEM((N, ncls), jnp.float32)],
        ),
        out_shape=jax.ShapeDtypeStruct((N, ncls), jnp.float32),
    )(im, jm, ym, adjn, y1, W2, b2.reshape(1, ncls))

    return (out, adjn)


# speedup vs baseline: 1.0138x; 1.0138x over previous
import jax
import jax.numpy as jnp
from jax.experimental import pallas as pl
from jax.experimental.pallas import tpu as pltpu

EOS = 1e-10
BM = 1024
BN = 1024
NSLOT = 3


def _elu1(a):
    return jnp.where(a > 0, a + 1.0, jnp.exp(a))


def _pair_maps(nb):
    im, jm = [], []
    for i in range(nb):
        for j in range(i, nb):
            im.append(i)
            jm.append(j)
    return jnp.array(im, jnp.int32), jnp.array(jm, jnp.int32)


def _pair_maps_main(nb):
    im, jm, ati, atj = [], [], [], []
    prev = None
    for i in range(nb):
        for (a, b) in [(i, j) for j in range(i + 1, nb)] + [(i, i)]:
            im.append(a)
            jm.append(b)
            cur = (b, a) if a != b else (prev if prev is not None else (a, b))
            ati.append(cur[0])
            atj.append(cur[1])
            prev = cur
    return (jnp.array(im, jnp.int32), jnp.array(jm, jnp.int32),
            jnp.array(ati, jnp.int32), jnp.array(atj, jnp.int32))


def _stats_kernel(
    a_ref, x_ref, w1_ref, b1_ref, dc_ref, dr_ref, h1_ref, degc_ref, degr_ref
):
    i = pl.program_id(0)
    j = pl.program_id(1)
    nbi = pl.num_programs(0)
    nbj = pl.num_programs(1)

    @pl.when((i == 0) & (j == 0))
    def _():
        degc_ref[:] = jnp.zeros_like(degc_ref)
        degr_ref[:] = jnp.zeros_like(degr_ref)

    e = _elu1(a_ref[:])
    rs = jnp.sum(e, axis=1, keepdims=True)
    cs = jnp.sum(e, axis=0, keepdims=True)
    degc_ref[pl.ds(i * BM, BM), :] += 0.5 * rs
    degc_ref[pl.ds(j * BN, BN), :] += 0.5 * cs.T
    degr_ref[:, pl.ds(i * BM, BM)] += 0.5 * rs.T
    degr_ref[:, pl.ds(j * BN, BN)] += 0.5 * cs

    @pl.when(j == 0)
    def _():
        h1_ref[:] = (
            jnp.dot(x_ref[:], w1_ref[:], preferred_element_type=jnp.float32)
            + b1_ref[:]
        )

    @pl.when((i == nbi - 1) & (j == nbj - 1))
    def _():
        dc_ref[:] = 1.0 / (jnp.sqrt(degc_ref[:]) + EOS)
        dr_ref[:] = 1.0 / (jnp.sqrt(degr_ref[:]) + EOS)


def _wait_pair_copies(im_ref, jm_ref, adjn_ref, ab_ref, tb_ref, sema_ref, semb_ref, kk):
    slot = jax.lax.rem(kk, NSLOT)
    i2 = im_ref[kk]
    j2 = jm_ref[kk]
    pltpu.make_async_copy(
        ab_ref.at[slot],
        adjn_ref.at[pl.ds(i2 * BM, BM), pl.ds(j2 * BN, BN)],
        sema_ref.at[slot],
    ).wait()

    @pl.when(i2 != j2)
    def _():
        pltpu.make_async_copy(
            tb_ref.at[slot],
            adjn_ref.at[pl.ds(j2 * BM, BM), pl.ds(i2 * BN, BN)],
            semb_ref.at[slot],
        ).wait()


def _main_kernel(
    im_ref, jm_ref, ati_ref, atj_ref,
    a_ref, at_ref, h1j_ref, h1i_ref, dc_ref, dr_ref,
    adjn_ref, y1_ref, ab_ref, tb_ref, sema_ref, semb_ref,
):
    k = pl.program_id(0)
    npairs = pl.num_programs(0)
    slot = jax.lax.rem(k, NSLOT)
    i = im_ref[k]
    j = jm_ref[k]

    @pl.when(k >= NSLOT)
    def _():
        _wait_pair_copies(
            im_ref, jm_ref, adjn_ref, ab_ref, tb_ref, sema_ref, semb_ref, k - NSLOT
        )

    @pl.when(k == 0)
    def _():
        y1_ref[:] = jnp.zeros_like(y1_ref)

    t = jnp.where(i == j, a_ref[:], at_ref[:])
    e = 0.5 * (_elu1(a_ref[:]) + _elu1(t).T)
    adjn = dc_ref[:] * e * dr_ref[:]
    ab_ref[slot] = adjn
    pltpu.make_async_copy(
        ab_ref.at[slot],
        adjn_ref.at[pl.ds(i * BM, BM), pl.ds(j * BN, BN)],
        sema_ref.at[slot],
    ).start()
    tb_ref[slot] = adjn.T

    @pl.when(i != j)
    def _():
        pltpu.make_async_copy(
            tb_ref.at[slot],
            adjn_ref.at[pl.ds(j * BM, BM), pl.ds(i * BN, BN)],
            semb_ref.at[slot],
        ).start()

    y1_ref[pl.ds(i * BM, BM), :] += jnp.dot(
        adjn, h1j_ref[:], preferred_element_type=jnp.float32
    )

    @pl.when(i != j)
    def _():
        y1_ref[pl.ds(j * BN, BN), :] += jax.lax.dot_general(
            adjn, h1i_ref[:],
            (((0,), (0,)), ((), ())),
            preferred_element_type=jnp.float32,
        )

    @pl.when(k == npairs - 1)
    def _():
        for d in range(NSLOT - 1, 0, -1):
            @pl.when(k >= d)
            def _(d=d):
                _wait_pair_copies(
                    im_ref, jm_ref, adjn_ref, ab_ref, tb_ref, sema_ref, semb_ref,
                    k - d,
                )

        _wait_pair_copies(
            im_ref, jm_ref, adjn_ref, ab_ref, tb_ref, sema_ref, semb_ref, k
        )


def _out_kernel(
    im_ref, jm_ref, ym_ref, adjn_ref, y1j_ref, w2_ref, b2_ref,
    out_ref, h2_ref,
):
    k = pl.program_id(0)
    i = im_ref[k]
    j = jm_ref[k]

    @pl.when(k == 0)
    def _():
        out_ref[:] = jnp.zeros_like(out_ref)

    @pl.when(i == 0)
    def _():
        h = jnp.maximum(y1j_ref[:], 0.0)
        h2_ref[pl.ds(j * BN, BN), :] = (
            jnp.dot(h, w2_ref[:], preferred_element_type=jnp.float32) + b2_ref[:]
        )

    out_ref[pl.ds(i * BM, BM), :] += jnp.dot(
        adjn_ref[:], h2_ref[pl.ds(j * BN, BN), :], preferred_element_type=jnp.float32
    )

    @pl.when(i != j)
    def _():
        out_ref[pl.ds(j * BN, BN), :] += jax.lax.dot_general(
            adjn_ref[:], h2_ref[pl.ds(i * BM, BM), :],
            (((0,), (0,)), ((), ())),
            preferred_element_type=jnp.float32,
        )


def kernel(features, x, Adj_param, W1, b1, W2, b2):
    N = Adj_param.shape[0]
    in_dim = x.shape[1]
    hid = W1.shape[1]
    ncls = W2.shape[1]
    nb = N // BM
    npairs = nb * (nb + 1) // 2
    im, jm = _pair_maps(nb)

    dc, dr, h1 = pl.pallas_call(
        _stats_kernel,
        grid=(nb, nb),
        in_specs=[
            pl.BlockSpec((BM, BN), lambda i, j: (i, j)),
            pl.BlockSpec((BM, in_dim), lambda i, j: (i, 0)),
            pl.BlockSpec((in_dim, hid), lambda i, j: (0, 0)),
            pl.BlockSpec((1, hid), lambda i, j: (0, 0)),
        ],
        out_specs=[
            pl.BlockSpec((N, 1), lambda i, j: (0, 0)),
            pl.BlockSpec((1, N), lambda i, j: (0, 0)),
            pl.BlockSpec((BM, hid), lambda i, j: (i, 0)),
        ],
        out_shape=[
            jax.ShapeDtypeStruct((N, 1), jnp.float32),
            jax.ShapeDtypeStruct((1, N), jnp.float32),
            jax.ShapeDtypeStruct((N, hid), jnp.float32),
        ],
        scratch_shapes=[
            pltpu.VMEM((N, 1), jnp.float32),
            pltpu.VMEM((1, N), jnp.float32),
        ],
    )(Adj_param, x, W1, b1.reshape(1, hid))

    imm, jmm, ati, atj = _pair_maps_main(nb)
    adjn, y1 = pl.pallas_call(
        _main_kernel,
        grid_spec=pltpu.PrefetchScalarGridSpec(
            num_scalar_prefetch=4,
            grid=(npairs,),
            in_specs=[
                pl.BlockSpec((BM, BN), lambda k, im, jm, ai, aj: (im[k], jm[k])),
                pl.BlockSpec((BN, BM), lambda k, im, jm, ai, aj: (ai[k], aj[k])),
                pl.BlockSpec((BN, hid), lambda k, im, jm, ai, aj: (jm[k], 0)),
                pl.BlockSpec((BM, hid), lambda k, im, jm, ai, aj: (im[k], 0)),
                pl.BlockSpec((BM, 1), lambda k, im, jm, ai, aj: (im[k], 0)),
                pl.BlockSpec((1, BN), lambda k, im, jm, ai, aj: (0, jm[k])),
            ],
            out_specs=[
                pl.BlockSpec(memory_space=pl.ANY),
                pl.BlockSpec((N, hid), lambda k, im, jm, ai, aj: (0, 0)),
            ],
            scratch_shapes=[
                pltpu.VMEM((NSLOT, BM, BN), jnp.float32),
                pltpu.VMEM((NSLOT, BN, BM), jnp.float32),
                pltpu.SemaphoreType.DMA((NSLOT,)),
                pltpu.SemaphoreType.DMA((NSLOT,)),
            ],
        ),
        out_shape=[
            jax.ShapeDtypeStruct((N, N), jnp.float32),
            jax.ShapeDtypeStruct((N, hid), jnp.float32),
        ],
    )(imm, jmm, ati, atj, Adj_param, Adj_param, h1, h1, dc, dr)

    ym = jnp.where(im == 0, jm, nb - 1)
    out = pl.pallas_call(
        _out_kernel,
        grid_spec=pltpu.PrefetchScalarGridSpec(
            num_scalar_prefetch=3,
            grid=(npairs,),
            in_specs=[
                pl.BlockSpec((BM, BN), lambda k, im, jm, ym: (im[k], jm[k])),
                pl.BlockSpec((BN, hid), lambda k, im, jm, ym: (ym[k], 0)),
                pl.BlockSpec((hid, ncls), lambda k, im, jm, ym: (0, 0)),
                pl.BlockSpec((1, ncls), lambda k, im, jm, ym: (0, 0)),
            ],
            out_specs=pl.BlockSpec((N, ncls), lambda k, im, jm, ym: (0, 0)),
            scratch_shapes=[pltpu.VMEM((N, ncls), jnp.float32)],
        ),
        out_shape=jax.ShapeDtypeStruct((N, ncls), jnp.float32),
    )(im, jm, ym, adjn, y1, W2, b2.reshape(1, ncls))

    return (out, adjn)


# speedup vs baseline: 1.2034x; 1.1871x over previous
import jax
import jax.numpy as jnp
from jax.experimental import pallas as pl
from jax.experimental.pallas import tpu as pltpu

EOS = 1e-10
BM = 1024
BN = 1024
NSLOT = 3


def _elu1(a):
    return jnp.where(a > 0, a + 1.0, jnp.exp(a))


def _pair_maps(nb):
    im, jm = [], []
    for i in range(nb):
        for j in range(i, nb):
            im.append(i)
            jm.append(j)
    return jnp.array(im, jnp.int32), jnp.array(jm, jnp.int32)


def _pair_maps_main(nb):
    im, jm, ati, atj, h1flag, esk = [], [], [], [], [], []
    prev = None
    for i in range(nb):
        row = [(i, j) for j in range(i + 1, nb)] + [(i, i)]
        for s, (a, b) in enumerate(row):
            im.append(a)
            jm.append(b)
            h1flag.append(1 if s == 0 else 0)
            esk.append(a * (2 * nb - a + 1) // 2 + (b - a))
            cur = (b, a) if a != b else (prev if prev is not None else (a, b))
            ati.append(cur[0])
            atj.append(cur[1])
            prev = cur
    return (jnp.array(im, jnp.int32), jnp.array(jm, jnp.int32),
            jnp.array(ati, jnp.int32), jnp.array(atj, jnp.int32),
            jnp.array(h1flag, jnp.int32), jnp.array(esk, jnp.int32))


def _stats_kernel(
    im_ref, jm_ref, ati_ref, atj_ref, hf_ref, esk_ref,
    a_ref, at_ref, x_ref, w1_ref, b1_ref,
    es_ref, dc_ref, dr_ref, h1_ref, degc_ref, degr_ref,
):
    k = pl.program_id(0)
    npairs = pl.num_programs(0)
    i = im_ref[k]
    j = jm_ref[k]

    @pl.when(k == 0)
    def _():
        degc_ref[:] = jnp.zeros_like(degc_ref)
        degr_ref[:] = jnp.zeros_like(degr_ref)

    t = jnp.where(i == j, a_ref[:], at_ref[:])
    esym = 0.5 * (_elu1(a_ref[:]) + _elu1(t).T)
    es_ref[:] = (esym - 1.0).astype(jnp.bfloat16).reshape(1, BM, BN)

    rs = jnp.sum(esym, axis=1, keepdims=True)
    degc_ref[pl.ds(i * BM, BM), :] += rs
    degr_ref[:, pl.ds(i * BM, BM)] += rs.T

    @pl.when(i != j)
    def _():
        cs = jnp.sum(esym, axis=0, keepdims=True)
        degc_ref[pl.ds(j * BN, BN), :] += cs.T
        degr_ref[:, pl.ds(j * BN, BN)] += cs

    @pl.when(hf_ref[k] == 1)
    def _():
        h1_ref[:] = (
            jnp.dot(x_ref[:], w1_ref[:], preferred_element_type=jnp.float32)
            + b1_ref[:]
        )

    @pl.when(k == npairs - 1)
    def _():
        dc_ref[:] = 1.0 / (jnp.sqrt(degc_ref[:]) + EOS)
        dr_ref[:] = 1.0 / (jnp.sqrt(degr_ref[:]) + EOS)


def _wait_pair_copies(im_ref, jm_ref, adjn_ref, ab_ref, tb_ref, sema_ref, semb_ref, kk):
    slot = jax.lax.rem(kk, NSLOT)
    i2 = im_ref[kk]
    j2 = jm_ref[kk]
    pltpu.make_async_copy(
        ab_ref.at[slot],
        adjn_ref.at[pl.ds(i2 * BM, BM), pl.ds(j2 * BN, BN)],
        sema_ref.at[slot],
    ).wait()

    @pl.when(i2 != j2)
    def _():
        pltpu.make_async_copy(
            tb_ref.at[slot],
            adjn_ref.at[pl.ds(j2 * BM, BM), pl.ds(i2 * BN, BN)],
            semb_ref.at[slot],
        ).wait()


def _main_kernel(
    im_ref, jm_ref,
    es_ref, h1_ref, dc_ref, dr_ref,
    adjn_ref, y1_ref, ab_ref, tb_ref, sema_ref, semb_ref,
):
    k = pl.program_id(0)
    npairs = pl.num_programs(0)
    slot = jax.lax.rem(k, NSLOT)
    i = im_ref[k]
    j = jm_ref[k]

    @pl.when(k >= NSLOT)
    def _():
        _wait_pair_copies(
            im_ref, jm_ref, adjn_ref, ab_ref, tb_ref, sema_ref, semb_ref, k - NSLOT
        )

    @pl.when(k == 0)
    def _():
        y1_ref[:] = jnp.zeros_like(y1_ref)

    dc = dc_ref[pl.ds(i * BM, BM), :]
    dr = dr_ref[:, pl.ds(j * BN, BN)]
    adjn = dc * (es_ref[0].astype(jnp.float32) + 1.0) * dr
    ab_ref[slot] = adjn
    pltpu.make_async_copy(
        ab_ref.at[slot],
        adjn_ref.at[pl.ds(i * BM, BM), pl.ds(j * BN, BN)],
        sema_ref.at[slot],
    ).start()
    tb_ref[slot] = adjn.T

    @pl.when(i != j)
    def _():
        pltpu.make_async_copy(
            tb_ref.at[slot],
            adjn_ref.at[pl.ds(j * BM, BM), pl.ds(i * BN, BN)],
            semb_ref.at[slot],
        ).start()

    y1_ref[pl.ds(i * BM, BM), :] += jnp.dot(
        adjn, h1_ref[pl.ds(j * BN, BN), :], preferred_element_type=jnp.float32
    )

    @pl.when(i != j)
    def _():
        y1_ref[pl.ds(j * BN, BN), :] += jax.lax.dot_general(
            adjn, h1_ref[pl.ds(i * BM, BM), :],
            (((0,), (0,)), ((), ())),
            preferred_element_type=jnp.float32,
        )

    @pl.when(k == npairs - 1)
    def _():
        for d in range(NSLOT - 1, 0, -1):
            @pl.when(k >= d)
            def _(d=d):
                _wait_pair_copies(
                    im_ref, jm_ref, adjn_ref, ab_ref, tb_ref, sema_ref, semb_ref,
                    k - d,
                )

        _wait_pair_copies(
            im_ref, jm_ref, adjn_ref, ab_ref, tb_ref, sema_ref, semb_ref, k
        )


def _out_kernel(
    im_ref, jm_ref, ym_ref,
    es_ref, dc_ref, dr_ref, y1j_ref, w2_ref, b2_ref,
    out_ref, h2_ref,
):
    k = pl.program_id(0)
    i = im_ref[k]
    j = jm_ref[k]

    @pl.when(k == 0)
    def _():
        out_ref[:] = jnp.zeros_like(out_ref)

    @pl.when(i == 0)
    def _():
        h = jnp.maximum(y1j_ref[:], 0.0)
        h2_ref[pl.ds(j * BN, BN), :] = (
            jnp.dot(h, w2_ref[:], preferred_element_type=jnp.float32) + b2_ref[:]
        )

    dc = dc_ref[pl.ds(i * BM, BM), :]
    dr = dr_ref[:, pl.ds(j * BN, BN)]
    adjn = dc * (es_ref[0].astype(jnp.float32) + 1.0) * dr

    out_ref[pl.ds(i * BM, BM), :] += jnp.dot(
        adjn, h2_ref[pl.ds(j * BN, BN), :], preferred_element_type=jnp.float32
    )

    @pl.when(i != j)
    def _():
        out_ref[pl.ds(j * BN, BN), :] += jax.lax.dot_general(
            adjn, h2_ref[pl.ds(i * BM, BM), :],
            (((0,), (0,)), ((), ())),
            preferred_element_type=jnp.float32,
        )


def kernel(features, x, Adj_param, W1, b1, W2, b2):
    N = Adj_param.shape[0]
    in_dim = x.shape[1]
    hid = W1.shape[1]
    ncls = W2.shape[1]
    nb = N // BM
    npairs = nb * (nb + 1) // 2
    im, jm = _pair_maps(nb)
    imm, jmm, ati, atj, h1f, esk_stats = _pair_maps_main(nb)

    esym, dc, dr, h1 = pl.pallas_call(
        _stats_kernel,
        grid_spec=pltpu.PrefetchScalarGridSpec(
            num_scalar_prefetch=6,
            grid=(npairs,),
            in_specs=[
                pl.BlockSpec((BM, BN), lambda k, im, jm, ai, aj, hf, ek: (im[k], jm[k])),
                pl.BlockSpec((BN, BM), lambda k, im, jm, ai, aj, hf, ek: (ai[k], aj[k])),
                pl.BlockSpec((BM, in_dim), lambda k, im, jm, ai, aj, hf, ek: (im[k], 0)),
                pl.BlockSpec((in_dim, hid), lambda k, im, jm, ai, aj, hf, ek: (0, 0)),
                pl.BlockSpec((1, hid), lambda k, im, jm, ai, aj, hf, ek: (0, 0)),
            ],
            out_specs=[
                pl.BlockSpec((1, BM, BN), lambda k, im, jm, ai, aj, hf, ek: (ek[k], 0, 0)),
                pl.BlockSpec((N, 1), lambda k, im, jm, ai, aj, hf, ek: (0, 0)),
                pl.BlockSpec((1, N), lambda k, im, jm, ai, aj, hf, ek: (0, 0)),
                pl.BlockSpec((BM, hid), lambda k, im, jm, ai, aj, hf, ek: (im[k], 0)),
            ],
            scratch_shapes=[
                pltpu.VMEM((N, 1), jnp.float32),
                pltpu.VMEM((1, N), jnp.float32),
            ],
        ),
        out_shape=[
            jax.ShapeDtypeStruct((npairs, BM, BN), jnp.bfloat16),
            jax.ShapeDtypeStruct((N, 1), jnp.float32),
            jax.ShapeDtypeStruct((1, N), jnp.float32),
            jax.ShapeDtypeStruct((N, hid), jnp.float32),
        ],
    )(imm, jmm, ati, atj, h1f, esk_stats, Adj_param, Adj_param, x, W1, b1.reshape(1, hid))

    adjn, y1 = pl.pallas_call(
        _main_kernel,
        grid_spec=pltpu.PrefetchScalarGridSpec(
            num_scalar_prefetch=2,
            grid=(npairs,),
            in_specs=[
                pl.BlockSpec((1, BM, BN), lambda k, im, jm: (k, 0, 0)),
                pl.BlockSpec((N, hid), lambda k, im, jm: (0, 0)),
                pl.BlockSpec((N, 1), lambda k, im, jm: (0, 0)),
                pl.BlockSpec((1, N), lambda k, im, jm: (0, 0)),
            ],
            out_specs=[
                pl.BlockSpec(memory_space=pl.ANY),
                pl.BlockSpec((N, hid), lambda k, im, jm: (0, 0)),
            ],
            scratch_shapes=[
                pltpu.VMEM((NSLOT, BM, BN), jnp.float32),
                pltpu.VMEM((NSLOT, BN, BM), jnp.float32),
                pltpu.SemaphoreType.DMA((NSLOT,)),
                pltpu.SemaphoreType.DMA((NSLOT,)),
            ],
        ),
        out_shape=[
            jax.ShapeDtypeStruct((N, N), jnp.float32),
            jax.ShapeDtypeStruct((N, hid), jnp.float32),
        ],
    )(im, jm, esym, h1, dc, dr)

    ym = jnp.where(im == 0, jm, nb - 1)
    out = pl.pallas_call(
        _out_kernel,
        grid_spec=pltpu.PrefetchScalarGridSpec(
            num_scalar_prefetch=3,
            grid=(npairs,),
            in_specs=[
                pl.BlockSpec((1, BM, BN), lambda k, im, jm, ym: (k, 0, 0)),
                pl.BlockSpec((N, 1), lambda k, im, jm, ym: (0, 0)),
                pl.BlockSpec((1, N), lambda k, im, jm, ym: (0, 0)),
                pl.BlockSpec((BN, hid), lambda k, im, jm, ym: (ym[k], 0)),
                pl.BlockSpec((hid, ncls), lambda k, im, jm, ym: (0, 0)),
                pl.BlockSpec((1, ncls), lambda k, im, jm, ym: (0, 0)),
            ],
            out_specs=pl.BlockSpec((N, ncls), lambda k, im, jm, ym: (0, 0)),
            scratch_shapes=[pltpu.VMEM((N, ncls), jnp.float32)],
        ),
        out_shape=jax.ShapeDtypeStruct((N, ncls), jnp.float32),
    )(im, jm, ym, esym, dc, dr, y1, W2, b2.reshape(1, ncls))

    return (out, adjn)
